# baseline (speedup 1.0000x reference)
import functools

import jax
import jax.numpy as jnp
from jax import lax
from jax.experimental import pallas as pl
from jax.experimental.pallas import tpu as pltpu
from jax.experimental.pallas import tpu_sc as plsc

T, N, E, D, H, OUT, B = 8, 10000, 160000, 256, 256, 128, 16
L = 5

NC, NS = 2, 16
NP = 10240
RB = 1024
GRID = NP // RB
HF = 128

ACH = 128
ACHT = 80
EROWS = NS * ACHT
EPAD = EROWS * ACH
TRASH = NP - 8
AROWS = NP // NS
NSLOT = 2
PH = 40

NW = NC * NS
DCHT = EROWS // NW
DW = 128


def _make_agg_kernel():
    def body(u2, src2, dst2, out, agg_sp, gidxb, dstb, r0, r1, g0, g1, s0, s1):
        c = lax.axis_index("c")
        s = lax.axis_index("s")
        rows = (r0, r1)
        gsem = (g0, g1)
        ssem = (s0, s1)

        def zr(i, _):
            for j in range(8):
                r0[i, pl.ds(j * 16, 16)] = jnp.zeros((16,), jnp.float32)
            return 0
        lax.fori_loop(0, ACH, zr, 0)
        for k in range(AROWS // ACH):
            pltpu.sync_copy(r0, agg_sp.at[pl.ds(s * AROWS + k * ACH, ACH), :])
        plsc.subcore_barrier()

        for p in range(ACHT // PH):
            base = s * ACHT + p * PH
            pltpu.sync_copy(src2.at[pl.ds(base, PH), :], gidxb)
            pltpu.sync_copy(dst2.at[pl.ds(base, PH), :], dstb)

            def fix(i, _):
                for j in range(8):
                    gidxb[i, pl.ds(j * 16, 16)] = (
                        gidxb[i, pl.ds(j * 16, 16)] * 2 + c)
                return 0
            lax.fori_loop(0, PH, fix, 0)

            def grp(g, _):
                for t in range(NSLOT):
                    k = g * NSLOT + t

                    @pl.when(g > 0)
                    def _():
                        pltpu.make_async_copy(
                            rows[t], agg_sp.at[dstb.at[k - NSLOT]],
                            ssem[t]).wait()

                    pltpu.async_copy(u2.at[gidxb.at[k]], rows[t],
                                     gsem[t]).wait()
                    pltpu.async_copy(rows[t], agg_sp.at[dstb.at[k]], ssem[t],
                                     add=True)
                return 0
            lax.fori_loop(0, PH // NSLOT, grp, 0)
            for t in range(NSLOT):
                k = PH - NSLOT + t
                pltpu.make_async_copy(rows[t], agg_sp.at[dstb.at[k]],
                                      ssem[t]).wait()
        plsc.subcore_barrier()

        for k in range(AROWS // ACH):
            off = s * AROWS + k * ACH
            pltpu.sync_copy(agg_sp.at[pl.ds(off, ACH), :],
                            out.at[pl.ds(c * NP + off, ACH), :])

    mesh = plsc.VectorSubcoreMesh(core_axis_name="c", subcore_axis_name="s",
                                  num_cores=NC, num_subcores=NS)
    return pl.kernel(
        body,
        out_type=jax.ShapeDtypeStruct((NC * NP, HF), jnp.float32),
        mesh=mesh,
        scratch_types=[
            pltpu.VMEM_SHARED((NP, HF), jnp.float32),
            pltpu.VMEM((PH, ACH), jnp.int32),
            pltpu.VMEM((PH, ACH), jnp.int32),
            pltpu.VMEM((ACH, HF), jnp.float32),
            pltpu.VMEM((ACH, HF), jnp.float32),
            pltpu.SemaphoreType.DMA,
            pltpu.SemaphoreType.DMA,
            pltpu.SemaphoreType.DMA,
            pltpu.SemaphoreType.DMA,
        ],
    )


def _make_deg_kernel():
    def body(dst2, out, deg_sp, ones, dstb, sem):
        c = lax.axis_index("c")
        s = lax.axis_index("s")
        w = c * NS + s

        def fill0(i, _):
            for j in range(DW // 16):
                ones[i, pl.ds(j * 16, 16)] = jnp.zeros((16,), jnp.float32)
            return 0
        lax.fori_loop(0, ACH, fill0, 0)
        pltpu.sync_copy(dst2.at[pl.ds(w * DCHT, DCHT), :], dstb)
        for k in range(AROWS // ACH):
            pltpu.sync_copy(ones, deg_sp.at[pl.ds(s * AROWS + k * ACH, ACH), :])

        def fill1(i, _):
            for j in range(DW // 16):
                ones[i, pl.ds(j * 16, 16)] = jnp.full((16,), 1.0, jnp.float32)
            return 0
        lax.fori_loop(0, ACH, fill1, 0)
        plsc.subcore_barrier()

        def chunk(i, _):
            pltpu.async_copy(ones, deg_sp.at[dstb.at[i]], sem, add=True)
            return 0
        lax.fori_loop(0, DCHT, chunk, 0)

        def drain(i, _):
            pltpu.make_async_copy(ones, deg_sp.at[dstb.at[0]], sem).wait()
            return 0
        lax.fori_loop(0, DCHT, drain, 0)
        plsc.subcore_barrier()

        for k in range(AROWS // ACH):
            off = s * AROWS + k * ACH
            pltpu.sync_copy(deg_sp.at[pl.ds(off, ACH), :],
                            out.at[pl.ds(c * NP + off, ACH), :])

    mesh = plsc.VectorSubcoreMesh(core_axis_name="c", subcore_axis_name="s",
                                  num_cores=NC, num_subcores=NS)
    return pl.kernel(
        body,
        out_type=jax.ShapeDtypeStruct((NC * NP, DW), jnp.float32),
        mesh=mesh,
        scratch_types=[
            pltpu.VMEM_SHARED((NP, DW), jnp.float32),
            pltpu.VMEM((ACH, DW), jnp.float32),
            pltpu.VMEM((DCHT, ACH), jnp.int32),
            pltpu.SemaphoreType.DMA,
        ],
    )


def _dinv_from(deg_ref):
    d = deg_ref[0, :, 0:1] + deg_ref[1, :, 0:1] + 1.0
    return lax.rsqrt(d)


def _a0_body(x_ref, deg_ref, w_ref, u_ref):
    dinv = _dinv_from(deg_ref)
    u_ref[...] = (x_ref[...] @ w_ref[...]) * dinv


def _b_body(agg_ref, u_ref, deg_ref, w_ref, b_ref, un_ref):
    dinv = _dinv_from(deg_ref)
    aggc = jnp.concatenate([agg_ref[0], agg_ref[1]], axis=1)
    x = jax.nn.relu(dinv * (aggc + u_ref[...]) + b_ref[...])
    un_ref[...] = (x @ w_ref[...]) * dinv


def _pool_body(agg_ref, u_ref, deg_ref, b_ref, pt_ref, out_ref, acc_ref):
    i = pl.program_id(0)
    dinv = _dinv_from(deg_ref)
    aggc = jnp.concatenate([agg_ref[0], agg_ref[1]], axis=1)
    x = jax.nn.relu(dinv * (aggc + u_ref[...]) + b_ref[...])
    xx = jnp.concatenate([x, jnp.ones((RB, HF), jnp.float32)], axis=1)
    part = lax.dot_general(pt_ref[...], xx, (((1,), (0,)), ((), ())),
                           preferred_element_type=jnp.float32)

    @pl.when(i == 0)
    def _():
        acc_ref[...] = jnp.zeros_like(acc_ref)

    acc_ref[...] += part

    @pl.when(i == GRID - 1)
    def _():
        ssum = acc_ref[:, :D]
        cnt = jnp.maximum(acc_ref[:, D:], 1.0)
        out_ref[...] = jnp.concatenate([ssum[:, :HF] / cnt, ssum[:, HF:] / cnt],
                                       axis=1)


def _lstm_body(xs_ref, wi_ref, wh_ref, b_ref, fw_ref, fb_ref, out_ref):
    h = jnp.zeros((B, H), jnp.float32)
    c = jnp.zeros((B, H), jnp.float32)
    wi = wi_ref[...]
    wh = wh_ref[...]
    bias = b_ref[...]
    for t in range(T):
        g = xs_ref[t] @ wi + h @ wh + bias
        ii = jax.nn.sigmoid(g[:, :H])
        ff = jax.nn.sigmoid(g[:, H:2 * H])
        gg = jnp.tanh(g[:, 2 * H:3 * H])
        oo = jax.nn.sigmoid(g[:, 3 * H:])
        c = ff * c + ii * gg
        h = oo * jnp.tanh(c)
    out_ref[...] = h @ fw_ref[...] + fb_ref[...]


_agg_call = None
_deg_call = None


def _get_sc_calls():
    global _agg_call, _deg_call
    if _agg_call is None:
        _agg_call = _make_agg_kernel()
        _deg_call = _make_deg_kernel()
    return _agg_call, _deg_call


_row = lambda i: (i, 0)
_deg_spec = pl.BlockSpec((2, RB, DW), lambda i: (0, i, 0))
_agg_spec = pl.BlockSpec((2, RB, HF), lambda i: (0, i, 0))
_full_spec = pl.BlockSpec((RB, D), _row)
_w_spec = pl.BlockSpec((D, H), lambda i: (0, 0))
_b_spec = pl.BlockSpec((1, H), lambda i: (0, 0))

_a0_call = pl.pallas_call(
    _a0_body,
    grid=(GRID,),
    in_specs=[_full_spec, _deg_spec, _w_spec],
    out_specs=_full_spec,
    out_shape=jax.ShapeDtypeStruct((NP, D), jnp.float32),
)

_b_call = pl.pallas_call(
    _b_body,
    grid=(GRID,),
    in_specs=[_agg_spec, _full_spec, _deg_spec, _w_spec, _b_spec],
    out_specs=_full_spec,
    out_shape=jax.ShapeDtypeStruct((NP, D), jnp.float32),
)

_pool_call = pl.pallas_call(
    _pool_body,
    grid=(GRID,),
    in_specs=[_agg_spec, _full_spec, _deg_spec, _b_spec,
              pl.BlockSpec((B, RB), lambda i: (0, i))],
    out_specs=pl.BlockSpec((B, D), lambda i: (0, 0)),
    out_shape=jax.ShapeDtypeStruct((B, D), jnp.float32),
    scratch_shapes=[pltpu.VMEM((B, D + HF), jnp.float32)],
)

_lstm_call = pl.pallas_call(
    _lstm_body,
    out_shape=jax.ShapeDtypeStruct((B, OUT), jnp.float32),
)


def kernel(x, edge_index, batch, conv_W, conv_b, lstm_Wi, lstm_Wh, lstm_b, fc_W, fc_b):
    agg_call, deg_call = _get_sc_calls()
    ei = edge_index.astype(jnp.int32)
    padn = EPAD - E
    src_all = jnp.concatenate(
        [ei[:, 0, :], jnp.zeros((T, padn), jnp.int32)], axis=1
    ).reshape(T, EROWS, ACH)
    dst_all = jnp.concatenate(
        [ei[:, 1, :], jnp.full((T, padn), TRASH, jnp.int32)], axis=1
    ).reshape(T, EROWS, ACH)
    bt = batch.astype(jnp.int32)
    xp = jnp.pad(x, ((0, 0), (0, NP - N), (0, 0)))
    gids = jnp.arange(B, dtype=jnp.int32)[:, None]
    bias_rows = conv_b[:, None, :]

    pooled = []
    for t in range(T):
        st, dt = src_all[t], dst_all[t]
        deg2 = deg_call(dt).reshape(2, NP, DW)
        u = _a0_call(xp[t], deg2, conv_W[0])
        for l in range(1, L):
            agg = agg_call(u.reshape(2 * NP, HF), st, dt).reshape(2, NP, HF)
            u = _b_call(agg, u, deg2, conv_W[l], bias_rows[l - 1])
        agg = agg_call(u.reshape(2 * NP, HF), st, dt).reshape(2, NP, HF)
        pt = (bt[t][None, :] == gids).astype(jnp.float32)
        pt = jnp.pad(pt, ((0, 0), (0, NP - N)))
        pooled.append(_pool_call(agg, u, deg2, bias_rows[L - 1], pt))

    xs = jnp.stack(pooled, axis=0)
    return _lstm_call(xs, lstm_Wi, lstm_Wh, lstm_b, fc_W, fc_b)


# speedup vs baseline: 1.0561x; 1.0561x over previous
import jax
import jax.numpy as jnp
from jax import lax
from jax.experimental import pallas as pl
from jax.experimental.pallas import tpu as pltpu
from jax.experimental.pallas import tpu_sc as plsc

T, N, E, D, H, OUT, B = 8, 10000, 160000, 256, 256, 128, 16
L = 5

NC, NS = 2, 16
NP = 10240
RB = 1024
GRID = NP // RB
HF = 128

ACH = 128
ACHT = 80
EPAD = NS * ACHT * ACH
TRASH = NP - 8
AROWS = NP // NS

NW = NC * NS
DCHT = EPAD // (NW * ACH)
DW = 128


def _make_agg_kernel():
    def body(u2, src1, dst1, out, agg_sp,
             ga, gb, da, db, ra, rb, sga, sgb):
        c = lax.axis_index("c")
        s = lax.axis_index("s")
        ebase = s * (ACHT * ACH)

        def zr(i, _):
            for j in range(8):
                ra[i, pl.ds(j * 16, 16)] = jnp.zeros((16,), jnp.float32)
            return 0
        lax.fori_loop(0, ACH, zr, 0)
        for k in range(AROWS // ACH):
            pltpu.sync_copy(ra, agg_sp.at[pl.ds(s * AROWS + k * ACH, ACH), :])
        plsc.subcore_barrier()

        def stage(k, gbuf, dbuf):
            eb = ebase + k * ACH
            pltpu.sync_copy(src1.at[pl.ds(eb, ACH)], gbuf)
            pltpu.sync_copy(dst1.at[pl.ds(eb, ACH)], dbuf)
            for j in range(8):
                gbuf[pl.ds(j * 16, 16)] = gbuf[pl.ds(j * 16, 16)] * 2 + c

        stage(0, ga, da)
        pltpu.async_copy(u2.at[ga], ra, sga)

        def grp(g, _):
            k = 2 * g
            stage(k + 1, gb, db)
            pltpu.async_copy(u2.at[gb], rb, sgb)
            pltpu.make_async_copy(u2.at[ga], ra, sga).wait()
            pltpu.sync_copy(ra, agg_sp.at[da], add=True)

            @pl.when(g < ACHT // 2 - 1)
            def _():
                stage(k + 2, ga, da)
                pltpu.async_copy(u2.at[ga], ra, sga)

            pltpu.make_async_copy(u2.at[gb], rb, sgb).wait()
            pltpu.sync_copy(rb, agg_sp.at[db], add=True)
            return 0
        lax.fori_loop(0, ACHT // 2, grp, 0)
        plsc.subcore_barrier()

        for k in range(AROWS // ACH):
            off = s * AROWS + k * ACH
            pltpu.sync_copy(agg_sp.at[pl.ds(off, ACH), :],
                            out.at[pl.ds(c * NP + off, ACH), :])

    mesh = plsc.VectorSubcoreMesh(core_axis_name="c", subcore_axis_name="s",
                                  num_cores=NC, num_subcores=NS)
    return pl.kernel(
        body,
        out_type=jax.ShapeDtypeStruct((NC * NP, HF), jnp.float32),
        mesh=mesh,
        scratch_types=[
            pltpu.VMEM_SHARED((NP, HF), jnp.float32),
            pltpu.VMEM((ACH,), jnp.int32),
            pltpu.VMEM((ACH,), jnp.int32),
            pltpu.VMEM((ACH,), jnp.int32),
            pltpu.VMEM((ACH,), jnp.int32),
            pltpu.VMEM((ACH, HF), jnp.float32),
            pltpu.VMEM((ACH, HF), jnp.float32),
            pltpu.SemaphoreType.DMA,
            pltpu.SemaphoreType.DMA,
        ],
    )


def _make_deg_kernel():
    def body(dst1, out, deg_sp, ones, da, db, sa, sb):
        c = lax.axis_index("c")
        s = lax.axis_index("s")
        w = c * NS + s
        ebase = w * (DCHT * ACH)

        def fill0(i, _):
            for j in range(DW // 16):
                ones[i, pl.ds(j * 16, 16)] = jnp.zeros((16,), jnp.float32)
            return 0
        lax.fori_loop(0, ACH, fill0, 0)
        for k in range(AROWS // ACH):
            pltpu.sync_copy(ones, deg_sp.at[pl.ds(s * AROWS + k * ACH, ACH), :])

        def fill1(i, _):
            for j in range(DW // 16):
                ones[i, pl.ds(j * 16, 16)] = jnp.full((16,), 1.0, jnp.float32)
            return 0
        lax.fori_loop(0, ACH, fill1, 0)
        plsc.subcore_barrier()

        def grp(g, _):
            k = 2 * g

            @pl.when(g > 0)
            def _():
                pltpu.make_async_copy(ones, deg_sp.at[da], sa).wait()
            pltpu.sync_copy(dst1.at[pl.ds(ebase + k * ACH, ACH)], da)
            pltpu.async_copy(ones, deg_sp.at[da], sa, add=True)

            @pl.when(g > 0)
            def _():
                pltpu.make_async_copy(ones, deg_sp.at[db], sb).wait()
            pltpu.sync_copy(dst1.at[pl.ds(ebase + (k + 1) * ACH, ACH)], db)
            pltpu.async_copy(ones, deg_sp.at[db], sb, add=True)
            return 0
        lax.fori_loop(0, DCHT // 2, grp, 0)
        pltpu.make_async_copy(ones, deg_sp.at[da], sa).wait()
        pltpu.make_async_copy(ones, deg_sp.at[db], sb).wait()
        plsc.subcore_barrier()

        for k in range(AROWS // ACH):
            off = s * AROWS + k * ACH
            pltpu.sync_copy(deg_sp.at[pl.ds(off, ACH), :],
                            out.at[pl.ds(c * NP + off, ACH), :])

    mesh = plsc.VectorSubcoreMesh(core_axis_name="c", subcore_axis_name="s",
                                  num_cores=NC, num_subcores=NS)
    return pl.kernel(
        body,
        out_type=jax.ShapeDtypeStruct((NC * NP, DW), jnp.float32),
        mesh=mesh,
        scratch_types=[
            pltpu.VMEM_SHARED((NP, DW), jnp.float32),
            pltpu.VMEM((ACH, DW), jnp.float32),
            pltpu.VMEM((ACH,), jnp.int32),
            pltpu.VMEM((ACH,), jnp.int32),
            pltpu.SemaphoreType.DMA,
            pltpu.SemaphoreType.DMA,
        ],
    )


def _dinv_from(deg_ref):
    d = deg_ref[0, :, 0:1] + deg_ref[1, :, 0:1] + 1.0
    return lax.rsqrt(d)


def _a0_body(x_ref, deg_ref, w_ref, u_ref):
    dinv = _dinv_from(deg_ref)
    u_ref[...] = (x_ref[...] @ w_ref[...]) * dinv


def _b_body(agg_ref, u_ref, deg_ref, w_ref, b_ref, un_ref):
    dinv = _dinv_from(deg_ref)
    aggc = jnp.concatenate([agg_ref[0], agg_ref[1]], axis=1)
    x = jax.nn.relu(dinv * (aggc + u_ref[...]) + b_ref[...])
    un_ref[...] = (x @ w_ref[...]) * dinv


def _pool_body(agg_ref, u_ref, deg_ref, b_ref, pt_ref, out_ref, acc_ref):
    i = pl.program_id(0)
    dinv = _dinv_from(deg_ref)
    aggc = jnp.concatenate([agg_ref[0], agg_ref[1]], axis=1)
    x = jax.nn.relu(dinv * (aggc + u_ref[...]) + b_ref[...])
    xx = jnp.concatenate([x, jnp.ones((RB, HF), jnp.float32)], axis=1)
    part = lax.dot_general(pt_ref[...], xx, (((1,), (0,)), ((), ())),
                           preferred_element_type=jnp.float32)

    @pl.when(i == 0)
    def _():
        acc_ref[...] = jnp.zeros_like(acc_ref)

    acc_ref[...] += part

    @pl.when(i == GRID - 1)
    def _():
        ssum = acc_ref[:, :D]
        cnt = jnp.maximum(acc_ref[:, D:], 1.0)
        out_ref[...] = jnp.concatenate([ssum[:, :HF] / cnt, ssum[:, HF:] / cnt],
                                       axis=1)


def _lstm_body(xs_ref, wi_ref, wh_ref, b_ref, fw_ref, fb_ref, out_ref):
    h = jnp.zeros((B, H), jnp.float32)
    c = jnp.zeros((B, H), jnp.float32)
    wi = wi_ref[...]
    wh = wh_ref[...]
    bias = b_ref[...]
    for t in range(T):
        g = xs_ref[t] @ wi + h @ wh + bias
        ii = jax.nn.sigmoid(g[:, :H])
        ff = jax.nn.sigmoid(g[:, H:2 * H])
        gg = jnp.tanh(g[:, 2 * H:3 * H])
        oo = jax.nn.sigmoid(g[:, 3 * H:])
        c = ff * c + ii * gg
        h = oo * jnp.tanh(c)
    out_ref[...] = h @ fw_ref[...] + fb_ref[...]


_agg_call = None
_deg_call = None


def _get_sc_calls():
    global _agg_call, _deg_call
    if _agg_call is None:
        _agg_call = _make_agg_kernel()
        _deg_call = _make_deg_kernel()
    return _agg_call, _deg_call


_row = lambda i: (i, 0)
_deg_spec = pl.BlockSpec((2, RB, DW), lambda i: (0, i, 0))
_agg_spec = pl.BlockSpec((2, RB, HF), lambda i: (0, i, 0))
_full_spec = pl.BlockSpec((RB, D), _row)
_w_spec = pl.BlockSpec((D, H), lambda i: (0, 0))
_b_spec = pl.BlockSpec((1, H), lambda i: (0, 0))

_a0_call = pl.pallas_call(
    _a0_body,
    grid=(GRID,),
    in_specs=[_full_spec, _deg_spec, _w_spec],
    out_specs=_full_spec,
    out_shape=jax.ShapeDtypeStruct((NP, D), jnp.float32),
)

_b_call = pl.pallas_call(
    _b_body,
    grid=(GRID,),
    in_specs=[_agg_spec, _full_spec, _deg_spec, _w_spec, _b_spec],
    out_specs=_full_spec,
    out_shape=jax.ShapeDtypeStruct((NP, D), jnp.float32),
)

_pool_call = pl.pallas_call(
    _pool_body,
    grid=(GRID,),
    in_specs=[_agg_spec, _full_spec, _deg_spec, _b_spec,
              pl.BlockSpec((B, RB), lambda i: (0, i))],
    out_specs=pl.BlockSpec((B, D), lambda i: (0, 0)),
    out_shape=jax.ShapeDtypeStruct((B, D), jnp.float32),
    scratch_shapes=[pltpu.VMEM((B, D + HF), jnp.float32)],
)

_lstm_call = pl.pallas_call(
    _lstm_body,
    out_shape=jax.ShapeDtypeStruct((B, OUT), jnp.float32),
)


def kernel(x, edge_index, batch, conv_W, conv_b, lstm_Wi, lstm_Wh, lstm_b, fc_W, fc_b):
    agg_call, deg_call = _get_sc_calls()
    ei = edge_index.astype(jnp.int32)
    padn = EPAD - E
    src_all = jnp.concatenate(
        [ei[:, 0, :], jnp.zeros((T, padn), jnp.int32)], axis=1)
    dst_all = jnp.concatenate(
        [ei[:, 1, :], jnp.full((T, padn), TRASH, jnp.int32)], axis=1)
    bt = batch.astype(jnp.int32)
    xp = jnp.pad(x, ((0, 0), (0, NP - N), (0, 0)))
    gids = jnp.arange(B, dtype=jnp.int32)[:, None]
    bias_rows = conv_b[:, None, :]

    pooled = []
    for t in range(T):
        st, dt = src_all[t], dst_all[t]
        deg2 = deg_call(dt).reshape(2, NP, DW)
        u = _a0_call(xp[t], deg2, conv_W[0])
        for l in range(1, L):
            agg = agg_call(u.reshape(2 * NP, HF), st, dt).reshape(2, NP, HF)
            u = _b_call(agg, u, deg2, conv_W[l], bias_rows[l - 1])
        agg = agg_call(u.reshape(2 * NP, HF), st, dt).reshape(2, NP, HF)
        pt = (bt[t][None, :] == gids).astype(jnp.float32)
        pt = jnp.pad(pt, ((0, 0), (0, NP - N)))
        pooled.append(_pool_call(agg, u, deg2, bias_rows[L - 1], pt))

    xs = jnp.stack(pooled, axis=0)
    return _lstm_call(xs, lstm_Wi, lstm_Wh, lstm_b, fc_W, fc_b)


# speedup vs baseline: 2.1107x; 1.9986x over previous
import jax
import jax.numpy as jnp
from jax import lax
from jax.experimental import pallas as pl
from jax.experimental.pallas import tpu as pltpu
from jax.experimental.pallas import tpu_sc as plsc

T, N, E, D, H, OUT, B = 8, 10000, 160000, 256, 256, 128, 16
L = 5

NC, NS = 2, 16
NP = 10240
RB = 1024
GRID = NP // RB
HF = 128

ACH = 128
ACHT = 80
EPAD = NS * ACHT * ACH
TRASH = NP - 8
AROWS = NP // NS

NW = NC * NS
DCHT = EPAD // (NW * ACH)
DW = 128


def _make_agg_kernel():
    def body(u2, src1, dst1, out, agg_sp,
             ga, gb, da, db, ra, rb, sga, sgb):
        c = lax.axis_index("c")
        s = lax.axis_index("s")
        ebase = s * (ACHT * ACH)

        def zr(i, _):
            for j in range(8):
                ra[i, pl.ds(j * 16, 16)] = jnp.zeros((16,), jnp.float32)
            return 0
        lax.fori_loop(0, ACH, zr, 0)
        for k in range(AROWS // ACH):
            pltpu.sync_copy(ra, agg_sp.at[pl.ds(s * AROWS + k * ACH, ACH), :])
        plsc.subcore_barrier()

        def stage(k, gbuf, dbuf):
            eb = ebase + k * ACH
            pltpu.sync_copy(src1.at[pl.ds(eb, ACH)], gbuf)
            pltpu.sync_copy(dst1.at[pl.ds(eb, ACH)], dbuf)
            for j in range(8):
                gbuf[pl.ds(j * 16, 16)] = gbuf[pl.ds(j * 16, 16)] * 2 + c

        stage(0, ga, da)
        pltpu.async_copy(u2.at[ga], ra, sga)

        def grp(g, _):
            k = 2 * g
            stage(k + 1, gb, db)
            pltpu.async_copy(u2.at[gb], rb, sgb)
            pltpu.make_async_copy(u2.at[ga], ra, sga).wait()
            pltpu.sync_copy(ra, agg_sp.at[da], add=True)

            @pl.when(g < ACHT // 2 - 1)
            def _():
                stage(k + 2, ga, da)
                pltpu.async_copy(u2.at[ga], ra, sga)

            pltpu.make_async_copy(u2.at[gb], rb, sgb).wait()
            pltpu.sync_copy(rb, agg_sp.at[db], add=True)
            return 0
        lax.fori_loop(0, ACHT // 2, grp, 0)
        plsc.subcore_barrier()

        for k in range(AROWS // ACH):
            off = s * AROWS + k * ACH
            pltpu.sync_copy(agg_sp.at[pl.ds(off, ACH), :],
                            out.at[pl.ds(c * NP + off, ACH), :])

    mesh = plsc.VectorSubcoreMesh(core_axis_name="c", subcore_axis_name="s",
                                  num_cores=NC, num_subcores=NS)
    return pl.kernel(
        body,
        out_type=jax.ShapeDtypeStruct((NC * NP, HF), jnp.float32),
        mesh=mesh,
        scratch_types=[
            pltpu.VMEM_SHARED((NP, HF), jnp.float32),
            pltpu.VMEM((ACH,), jnp.int32),
            pltpu.VMEM((ACH,), jnp.int32),
            pltpu.VMEM((ACH,), jnp.int32),
            pltpu.VMEM((ACH,), jnp.int32),
            pltpu.VMEM((ACH, HF), jnp.float32),
            pltpu.VMEM((ACH, HF), jnp.float32),
            pltpu.SemaphoreType.DMA,
            pltpu.SemaphoreType.DMA,
        ],
    )


def _make_deg_kernel():
    def body(dst1, out, deg_sp, ones, da, db, sa, sb):
        c = lax.axis_index("c")
        s = lax.axis_index("s")
        w = c * NS + s
        ebase = w * (DCHT * ACH)

        def fill0(i, _):
            for j in range(DW // 16):
                ones[i, pl.ds(j * 16, 16)] = jnp.zeros((16,), jnp.float32)
            return 0
        lax.fori_loop(0, ACH, fill0, 0)
        for k in range(AROWS // ACH):
            pltpu.sync_copy(ones, deg_sp.at[pl.ds(s * AROWS + k * ACH, ACH), :])

        def fill1(i, _):
            for j in range(DW // 16):
                ones[i, pl.ds(j * 16, 16)] = jnp.full((16,), 1.0, jnp.float32)
            return 0
        lax.fori_loop(0, ACH, fill1, 0)
        plsc.subcore_barrier()

        def grp(g, _):
            k = 2 * g

            @pl.when(g > 0)
            def _():
                pltpu.make_async_copy(ones, deg_sp.at[da], sa).wait()
            pltpu.sync_copy(dst1.at[pl.ds(ebase + k * ACH, ACH)], da)
            pltpu.async_copy(ones, deg_sp.at[da], sa, add=True)

            @pl.when(g > 0)
            def _():
                pltpu.make_async_copy(ones, deg_sp.at[db], sb).wait()
            pltpu.sync_copy(dst1.at[pl.ds(ebase + (k + 1) * ACH, ACH)], db)
            pltpu.async_copy(ones, deg_sp.at[db], sb, add=True)
            return 0
        lax.fori_loop(0, DCHT // 2, grp, 0)
        pltpu.make_async_copy(ones, deg_sp.at[da], sa).wait()
        pltpu.make_async_copy(ones, deg_sp.at[db], sb).wait()
        plsc.subcore_barrier()

        for k in range(AROWS // ACH):
            off = s * AROWS + k * ACH
            pltpu.sync_copy(deg_sp.at[pl.ds(off, ACH), :],
                            out.at[pl.ds(c * NP + off, ACH), :])

    mesh = plsc.VectorSubcoreMesh(core_axis_name="c", subcore_axis_name="s",
                                  num_cores=NC, num_subcores=NS)
    return pl.kernel(
        body,
        out_type=jax.ShapeDtypeStruct((NC * NP, DW), jnp.float32),
        mesh=mesh,
        scratch_types=[
            pltpu.VMEM_SHARED((NP, DW), jnp.float32),
            pltpu.VMEM((ACH, DW), jnp.float32),
            pltpu.VMEM((ACH,), jnp.int32),
            pltpu.VMEM((ACH,), jnp.int32),
            pltpu.SemaphoreType.DMA,
            pltpu.SemaphoreType.DMA,
        ],
    )


def _dinv_from(deg_ref):
    d = deg_ref[0, :, 0:1] + deg_ref[1, :, 0:1] + 1.0
    return lax.rsqrt(d)


def _a0_body(x_ref, deg_ref, w_ref, u_ref):
    dinv = _dinv_from(deg_ref)
    u_ref[...] = (x_ref[...] @ w_ref[...]) * dinv


def _b_body(agg_ref, u_ref, deg_ref, w_ref, b_ref, un_ref):
    dinv = _dinv_from(deg_ref)
    aggc = jnp.concatenate([agg_ref[0], agg_ref[1]], axis=1)
    x = jax.nn.relu(dinv * (aggc + u_ref[...]) + b_ref[...])
    un_ref[...] = (x @ w_ref[...]) * dinv


def _pool_body(agg_ref, u_ref, deg_ref, b_ref, pt_ref, out_ref, acc_ref):
    i = pl.program_id(0)
    dinv = _dinv_from(deg_ref)
    aggc = jnp.concatenate([agg_ref[0], agg_ref[1]], axis=1)
    x = jax.nn.relu(dinv * (aggc + u_ref[...]) + b_ref[...])
    xx = jnp.concatenate([x, jnp.ones((RB, HF), jnp.float32)], axis=1)
    part = lax.dot_general(pt_ref[...], xx, (((1,), (0,)), ((), ())),
                           preferred_element_type=jnp.float32)

    @pl.when(i == 0)
    def _():
        acc_ref[...] = jnp.zeros_like(acc_ref)

    acc_ref[...] += part

    @pl.when(i == GRID - 1)
    def _():
        ssum = acc_ref[:, :D]
        cnt = jnp.maximum(acc_ref[:, D:], 1.0)
        out_ref[...] = jnp.concatenate([ssum[:, :HF] / cnt, ssum[:, HF:] / cnt],
                                       axis=1)


def _lstm_body(xs_ref, wi_ref, wh_ref, b_ref, fw_ref, fb_ref, out_ref):
    h = jnp.zeros((B, H), jnp.float32)
    c = jnp.zeros((B, H), jnp.float32)
    wi = wi_ref[...]
    wh = wh_ref[...]
    bias = b_ref[...]
    for t in range(T):
        g = xs_ref[t] @ wi + h @ wh + bias
        ii = jax.nn.sigmoid(g[:, :H])
        ff = jax.nn.sigmoid(g[:, H:2 * H])
        gg = jnp.tanh(g[:, 2 * H:3 * H])
        oo = jax.nn.sigmoid(g[:, 3 * H:])
        c = ff * c + ii * gg
        h = oo * jnp.tanh(c)
    out_ref[...] = h @ fw_ref[...] + fb_ref[...]


_agg_call = None
_deg_call = None


def _get_sc_calls():
    global _agg_call, _deg_call
    if _agg_call is None:
        _agg_call = _make_agg_kernel()
        _deg_call = _make_deg_kernel()
    return _agg_call, _deg_call


_row = lambda i: (i, 0)
_deg_spec = pl.BlockSpec((2, RB, DW), lambda i: (0, i, 0))
_agg_spec = pl.BlockSpec((2, RB, HF), lambda i: (0, i, 0))
_full_spec = pl.BlockSpec((RB, D), _row)
_w_spec = pl.BlockSpec((D, H), lambda i: (0, 0))
_b_spec = pl.BlockSpec((1, H), lambda i: (0, 0))

_a0_call = pl.pallas_call(
    _a0_body,
    grid=(GRID,),
    in_specs=[_full_spec, _deg_spec, _w_spec],
    out_specs=_full_spec,
    out_shape=jax.ShapeDtypeStruct((NP, D), jnp.float32),
)

_b_call = pl.pallas_call(
    _b_body,
    grid=(GRID,),
    in_specs=[_agg_spec, _full_spec, _deg_spec, _w_spec, _b_spec],
    out_specs=_full_spec,
    out_shape=jax.ShapeDtypeStruct((NP, D), jnp.float32),
)

_pool_call = pl.pallas_call(
    _pool_body,
    grid=(GRID,),
    in_specs=[_agg_spec, _full_spec, _deg_spec, _b_spec,
              pl.BlockSpec((B, RB), lambda i: (0, i))],
    out_specs=pl.BlockSpec((B, D), lambda i: (0, 0)),
    out_shape=jax.ShapeDtypeStruct((B, D), jnp.float32),
    scratch_shapes=[pltpu.VMEM((B, D + HF), jnp.float32)],
)

_lstm_call = pl.pallas_call(
    _lstm_body,
    out_shape=jax.ShapeDtypeStruct((B, OUT), jnp.float32),
)


def kernel(x, edge_index, batch, conv_W, conv_b, lstm_Wi, lstm_Wh, lstm_b, fc_W, fc_b):
    agg_call, deg_call = _get_sc_calls()
    ei = edge_index.astype(jnp.int32)
    padn = EPAD - E
    pad_src = (jnp.arange(padn, dtype=jnp.int32) % N)[None, :]
    pad_dst = (N + jnp.arange(padn, dtype=jnp.int32) % (NP - N))[None, :]
    src_all = jnp.concatenate(
        [ei[:, 0, :], jnp.broadcast_to(pad_src, (T, padn))], axis=1)
    dst_all = jnp.concatenate(
        [ei[:, 1, :], jnp.broadcast_to(pad_dst, (T, padn))], axis=1)
    bt = batch.astype(jnp.int32)
    xp = jnp.pad(x, ((0, 0), (0, NP - N), (0, 0)))
    gids = jnp.arange(B, dtype=jnp.int32)[:, None]
    bias_rows = conv_b[:, None, :]

    pooled = []
    for t in range(T):
        st, dt = src_all[t], dst_all[t]
        deg2 = deg_call(dt).reshape(2, NP, DW)
        u = _a0_call(xp[t], deg2, conv_W[0])
        for l in range(1, L):
            agg = agg_call(u.reshape(2 * NP, HF), st, dt).reshape(2, NP, HF)
            u = _b_call(agg, u, deg2, conv_W[l], bias_rows[l - 1])
        agg = agg_call(u.reshape(2 * NP, HF), st, dt).reshape(2, NP, HF)
        pt = (bt[t][None, :] == gids).astype(jnp.float32)
        pt = jnp.pad(pt, ((0, 0), (0, NP - N)))
        pooled.append(_pool_call(agg, u, deg2, bias_rows[L - 1], pt))

    xs = jnp.stack(pooled, axis=0)
    return _lstm_call(xs, lstm_Wi, lstm_Wh, lstm_b, fc_W, fc_b)
